# baseline (speedup 1.0000x reference)
import jax
import jax.numpy as jnp
from jax import lax
from jax.experimental import pallas as pl
from jax.experimental.pallas import tpu as pltpu
from jax.experimental.pallas import tpu_sc as plsc

_N = 2_000_000
_NTYPES = 5
_NW = 32
_CE = 2000
_CHUNKS = _N // _CE
_ITERS = (_CHUNKS + _NW - 1) // _NW


def _sc_body(elems, types16, w0, w1, types_v, inbuf, w0buf, w1buf):
    c = lax.axis_index("c")
    s = lax.axis_index("s")
    wid = s * 2 + c
    pltpu.sync_copy(types16, types_v)
    tvs = [plsc.load_gather(types_v, [jnp.full((16,), a, jnp.int32)])
           for a in range(_NTYPES)]
    one = jnp.int32(1)
    zero = jnp.int32(0)

    def chunk_body(j, carry):
        ch = wid + j * _NW

        @pl.when(ch < _CHUNKS)
        def _():
            base = ch * _CE
            pltpu.sync_copy(elems.at[pl.ds(base, _CE)], inbuf)
            for q in range(_CE // 16):
                e = inbuf[pl.ds(16 * q, 16)]
                v = jnp.where(e == tvs[0], one, zero)
                v = v | jnp.where(e == tvs[1], jnp.int32(1 << 8), zero)
                v = v | jnp.where(e == tvs[2], jnp.int32(1 << 16), zero)
                v = v | jnp.where(e == tvs[3], jnp.int32(1 << 24), zero)
                w0buf[pl.ds(16 * q, 16)] = v
                w1buf[pl.ds(16 * q, 16)] = jnp.where(e == tvs[4], one, zero)
            pltpu.sync_copy(w0buf, w0.at[pl.ds(base, _CE)])
            pltpu.sync_copy(w1buf, w1.at[pl.ds(base, _CE)])

        return carry

    lax.fori_loop(0, _ITERS, chunk_body, 0)


def kernel(elems, atom_types):
    types16 = jnp.concatenate(
        [atom_types, jnp.full((11,), -1, atom_types.dtype)])
    mesh = plsc.VectorSubcoreMesh(core_axis_name="c", subcore_axis_name="s")
    w0, w1 = pl.kernel(
        _sc_body,
        out_type=[jax.ShapeDtypeStruct((_N,), jnp.int32),
                  jax.ShapeDtypeStruct((_N,), jnp.int32)],
        mesh=mesh,
        scratch_types=[
            pltpu.VMEM((16,), jnp.int32),
            pltpu.VMEM((_CE,), jnp.int32),
            pltpu.VMEM((_CE,), jnp.int32),
            pltpu.VMEM((_CE,), jnp.int32),
        ],
        compiler_params=pltpu.CompilerParams(needs_layout_passes=False),
    )(elems, types16)
    shifts = jnp.array([0, 8, 16, 24, 0], jnp.int32)[None, :]
    src = jnp.where(jnp.arange(5)[None, :] < 4, w0[:, None], w1[:, None])
    return ((src >> shifts) & 1).astype(jnp.bool_)
